# baseline (speedup 1.0000x reference)
import functools

import jax
import jax.numpy as jnp
from jax import lax
from jax.experimental import pallas as pl
from jax.experimental.pallas import tpu as pltpu

S = 128
N = 4096
D = 64


def _isir_body(tc_ref, wc_ref, prop_ref, gum_ref, m_ref,
               traj_out, w_out, i_out, r_out, best_ref):
    s = pl.program_id(0)
    u = gum_ref[0]
    g = -jnp.log(-jnp.log(u + 1e-12) + 1e-12)
    c_idx = lax.broadcasted_iota(jnp.int32, (1, N), 1)

    @pl.when(s == 0)
    def _():
        t = tc_ref[...]
        traj_out[0] = t
        w = wc_ref[...]
        w_out[0] = w
        best_ref[...] = w + g
        i_out[...] = jnp.zeros((1, N), jnp.int32)
        r_out[...] = c_idx

    @pl.when(s > 0)
    def _():
        t = prop_ref[0]
        traj_out[0] = t
        m = m_ref[...]
        half_m2 = 0.5 * jnp.sum(m * m)
        w = lax.dot_general(m, t, (((1,), (1,)), ((), ())),
                            preferred_element_type=jnp.float32,
                            precision=lax.Precision.HIGHEST) - half_m2
        w_out[0] = w
        score = w + g
        win = score > best_ref[...]
        best_ref[...] = jnp.where(win, score, best_ref[...])
        i_out[...] = jnp.where(win, s, i_out[...])
        r_out[...] = jnp.where(win, s * N + c_idx, r_out[...])


def _isir_pass(traj_cur, wc2, proposals, gu3, m2, interpret=False):
    return pl.pallas_call(
        _isir_body,
        grid=(S,),
        in_specs=[
            pl.BlockSpec((N, D), lambda s: (0, 0)),
            pl.BlockSpec((1, N), lambda s: (0, 0)),
            pl.BlockSpec((1, N, D), lambda s: (jnp.maximum(s - 1, 0), 0, 0)),
            pl.BlockSpec((1, 1, N), lambda s: (s, 0, 0)),
            pl.BlockSpec((1, D), lambda s: (0, 0)),
        ],
        out_specs=[
            pl.BlockSpec((1, N, D), lambda s: (s, 0, 0)),
            pl.BlockSpec((1, 1, N), lambda s: (s, 0, 0)),
            pl.BlockSpec((1, N), lambda s: (0, 0)),
            pl.BlockSpec((1, N), lambda s: (0, 0)),
        ],
        out_shape=[
            jax.ShapeDtypeStruct((S, N, D), jnp.float32),
            jax.ShapeDtypeStruct((S, 1, N), jnp.float32),
            jax.ShapeDtypeStruct((1, N), jnp.int32),
            jax.ShapeDtypeStruct((1, N), jnp.int32),
        ],
        scratch_shapes=[pltpu.VMEM((1, N), jnp.float32)],
        interpret=interpret,
    )(traj_cur, wc2, proposals, gu3, m2)


_NC, _NS, _L = 2, 16, 16
_NW = _NC * _NS
_BPW = N // _NW


@functools.cache
def _sc_gather_build():
    from jax.experimental.pallas import tpu_sc as plsc
    mesh = plsc.VectorSubcoreMesh(core_axis_name="c", subcore_axis_name="s")

    @functools.partial(
        pl.kernel, mesh=mesh,
        compiler_params=pltpu.CompilerParams(use_tc_tiling_on_sc=False),
        out_type=jax.ShapeDtypeStruct((N, D), jnp.float32),
        scratch_types=[
            pltpu.VMEM((_BPW,), jnp.int32),
            pltpu.VMEM((_BPW, D), jnp.float32),
            pltpu.SemaphoreType.DMA,
        ],
    )
    def gather_k(table_hbm, idx_hbm, out_hbm, idx_v, rows_v, sem):
        wid = lax.axis_index("s") * _NC + lax.axis_index("c")
        base = wid * _BPW
        pltpu.sync_copy(idx_hbm.at[pl.ds(base, _BPW)], idx_v)
        pltpu.async_copy(table_hbm.at[idx_v], rows_v, sem).wait()
        pltpu.sync_copy(rows_v, out_hbm.at[pl.ds(base, _BPW)])

    return gather_k


def kernel(traj_cur, weights_cur, proposals, gumbel_u, target_mean):
    wc2 = weights_cur.reshape(1, N)
    gu3 = gumbel_u.reshape(S, 1, N)
    m2 = target_mean.reshape(1, D)
    traj_tot, w3, i2, r2 = _isir_pass(traj_cur, wc2, proposals, gu3, m2)
    weights_tot = w3.reshape(S, N)
    i = i2.reshape(N)
    traj_sel = _sc_gather_build()(traj_tot.reshape(S * N, D), r2.reshape(N))
    return (traj_tot, weights_tot, i, traj_sel)


# speedup vs baseline: 5.3107x; 5.3107x over previous
import jax
import jax.numpy as jnp
from jax import lax
from jax.experimental import pallas as pl
from jax.experimental.pallas import tpu as pltpu

S = 128
N = 4096
D = 64


def _isir_body(tc_ref, wc_ref, prop_ref, gum_ref, m_ref,
               traj_out, w_out, i_out, sel_out, best_ref):
    s = pl.program_id(0)
    u = gum_ref[0]
    g = -jnp.log(-jnp.log(u + 1e-12) + 1e-12)

    @pl.when(s == 0)
    def _():
        t = tc_ref[...]
        traj_out[0] = t
        w = wc_ref[...]
        w_out[0] = w
        best_ref[...] = w + g
        i_out[...] = jnp.zeros((1, N), jnp.int32)
        sel_out[...] = t

    @pl.when(s > 0)
    def _():
        t = prop_ref[0]
        traj_out[0] = t
        m = m_ref[...]
        half_m2 = 0.5 * jnp.sum(m * m)
        w = jnp.sum(t * m, axis=0)[None, :] - half_m2
        w_out[0] = w
        score = w + g
        win = score > best_ref[...]
        best_ref[...] = jnp.where(win, score, best_ref[...])
        i_out[...] = jnp.where(win, s, i_out[...])

        @pl.when(jnp.any(win))
        def _():
            sel_out[...] = jnp.where(win, t, sel_out[...])


def _isir_pass(tct, wc2, props_t, gu3, m2):
    return pl.pallas_call(
        _isir_body,
        grid=(S,),
        in_specs=[
            pl.BlockSpec((D, N), lambda s: (0, 0)),
            pl.BlockSpec((1, N), lambda s: (0, 0)),
            pl.BlockSpec((1, D, N), lambda s: (jnp.maximum(s - 1, 0), 0, 0)),
            pl.BlockSpec((1, 1, N), lambda s: (s, 0, 0)),
            pl.BlockSpec((D, 1), lambda s: (0, 0)),
        ],
        out_specs=[
            pl.BlockSpec((1, D, N), lambda s: (s, 0, 0)),
            pl.BlockSpec((1, 1, N), lambda s: (s, 0, 0)),
            pl.BlockSpec((1, N), lambda s: (0, 0)),
            pl.BlockSpec((D, N), lambda s: (0, 0)),
        ],
        out_shape=[
            jax.ShapeDtypeStruct((S, D, N), jnp.float32),
            jax.ShapeDtypeStruct((S, 1, N), jnp.float32),
            jax.ShapeDtypeStruct((1, N), jnp.int32),
            jax.ShapeDtypeStruct((D, N), jnp.float32),
        ],
        scratch_shapes=[pltpu.VMEM((1, N), jnp.float32)],
    )(tct, wc2, props_t, gu3, m2)


def kernel(traj_cur, weights_cur, proposals, gumbel_u, target_mean):
    tct = traj_cur.T
    wc2 = weights_cur.reshape(1, N)
    props_t = proposals.transpose(0, 2, 1)
    gu3 = gumbel_u.reshape(S, 1, N)
    m2 = target_mean.reshape(D, 1)
    traj_tot_t, w3, i2, sel_t = _isir_pass(tct, wc2, props_t, gu3, m2)
    traj_tot = traj_tot_t.transpose(0, 2, 1)
    weights_tot = w3.reshape(S, N)
    i = i2.reshape(N)
    traj_sel = sel_t.T
    return (traj_tot, weights_tot, i, traj_sel)
